# baseline (speedup 1.0000x reference)
import functools

import jax
import jax.numpy as jnp
import numpy as np
from jax.experimental import pallas as pl
from jax.experimental.pallas import tpu as pltpu

D = 768
H = 12
HKV = 3
HPG = H // HKV
HD = 64
L = 32
STRIDE = 16
TOPN = 16
WIN = 512
S = 2048
NCMP = (S - L) // STRIDE + 1
NCMP_PAD = 128
NBLK = S // L
SCALE = 1.0 / np.sqrt(HD)

F32 = jnp.float32


def _ln(xb, g, b):
    m = jnp.mean(xb, axis=-1, keepdims=True)
    v = jnp.var(xb, axis=-1, keepdims=True)
    return (xb - m) * jax.lax.rsqrt(v + 1e-5) * g + b


def _k1_body(x_ref, g_ref, b_ref, w_ref, bc_ref, q_ref, k_ref, v_ref, gt_ref):
    xb = x_ref[:]
    ln = _ln(xb, g_ref[:], b_ref[:])
    out = jnp.dot(ln, w_ref[:], preferred_element_type=F32) + bc_ref[:]
    q_ref[:] = out[:, :D]
    k_ref[:] = out[:, D:D + HKV * HD]
    v_ref[:] = out[:, D + HKV * HD:D + 2 * HKV * HD]
    gt_ref[:] = jax.nn.sigmoid(out[:, D + 2 * HKV * HD:])


def _k1(x, ln1_g, ln1_b, Wcat, bcat):
    blk = 256
    return pl.pallas_call(
        _k1_body,
        grid=(S // blk,),
        compiler_params=pltpu.CompilerParams(dimension_semantics=("parallel",)),
        in_specs=[
            pl.BlockSpec((blk, D), lambda i: (i, 0)),
            pl.BlockSpec((1, D), lambda i: (0, 0)),
            pl.BlockSpec((1, D), lambda i: (0, 0)),
            pl.BlockSpec(Wcat.shape, lambda i: (0, 0)),
            pl.BlockSpec((1, Wcat.shape[1]), lambda i: (0, 0)),
        ],
        out_specs=[
            pl.BlockSpec((blk, D), lambda i: (i, 0)),
            pl.BlockSpec((blk, HKV * HD), lambda i: (i, 0)),
            pl.BlockSpec((blk, HKV * HD), lambda i: (i, 0)),
            pl.BlockSpec((blk, 128), lambda i: (i, 0)),
        ],
        out_shape=[
            jax.ShapeDtypeStruct((S, D), F32),
            jax.ShapeDtypeStruct((S, HKV * HD), F32),
            jax.ShapeDtypeStruct((S, HKV * HD), F32),
            jax.ShapeDtypeStruct((S, 128), F32),
        ],
    )(x, ln1_g, ln1_b, Wcat, bcat)


def _k2_body(kf_ref, vf_ref, wk_ref, bk_ref, wv_ref, bv_ref, ck_ref, cv_ref):
    kr = kf_ref[0]
    vr = vf_ref[0]
    zero = jnp.zeros((1, HD), F32)

    def proj(r, w_ref, b_ref):
        t0 = jnp.dot(r, w_ref[:STRIDE * HD], preferred_element_type=F32)
        t1 = jnp.dot(r, w_ref[STRIDE * HD:], preferred_element_type=F32)
        t1s = jnp.concatenate([t1[1:], zero], axis=0)
        return t0 + t1s + b_ref[:]

    ck_ref[0] = proj(kr, wk_ref, bk_ref)
    cv_ref[0] = proj(vr, wv_ref, bv_ref)


def _k2(kflat, vflat, Wck, bck, Wcv, bcv):
    return pl.pallas_call(
        _k2_body,
        grid=(HKV,),
        compiler_params=pltpu.CompilerParams(dimension_semantics=("parallel",)),
        in_specs=[
            pl.BlockSpec((1, S // STRIDE, STRIDE * HD), lambda g: (g, 0, 0)),
            pl.BlockSpec((1, S // STRIDE, STRIDE * HD), lambda g: (g, 0, 0)),
            pl.BlockSpec(Wck.shape, lambda g: (0, 0)),
            pl.BlockSpec((1, HD), lambda g: (0, 0)),
            pl.BlockSpec(Wcv.shape, lambda g: (0, 0)),
            pl.BlockSpec((1, HD), lambda g: (0, 0)),
        ],
        out_specs=[
            pl.BlockSpec((1, NCMP_PAD, HD), lambda g: (g, 0, 0)),
            pl.BlockSpec((1, NCMP_PAD, HD), lambda g: (g, 0, 0)),
        ],
        out_shape=[
            jax.ShapeDtypeStruct((HKV, NCMP_PAD, HD), F32),
            jax.ShapeDtypeStruct((HKV, NCMP_PAD, HD), F32),
        ],
    )(kflat, vflat, Wck, bck, Wcv, bcv)


QC3 = 512


def _k3_body(q_ref, ck_ref, cv_ref, out_ref, impq_ref):
    i = pl.program_id(1)
    ckm = ck_ref[0]
    cvm = cv_ref[0]
    qpos = i * QC3 + jax.lax.broadcasted_iota(jnp.int32, (QC3, 1), 0)
    nidx = jax.lax.broadcasted_iota(jnp.int32, (1, NCMP_PAD), 1)
    cmp_end = nidx * STRIDE + (L - 1)
    mask = qpos >= cmp_end
    pad = nidx < NCMP

    cps = jnp.zeros((QC3, NCMP_PAD), F32)
    for hp in range(HPG):
        qh = q_ref[:, hp * HD:(hp + 1) * HD]
        s = jax.lax.dot_general(qh, ckm, (((1,), (1,)), ((), ())),
                                preferred_element_type=F32) * SCALE
        s = jnp.where(mask, s, -1e9)
        s = jnp.where(pad, s, -1e30)
        m = jnp.max(s, axis=-1, keepdims=True)
        p = jnp.exp(s - m)
        cp = p / jnp.sum(p, axis=-1, keepdims=True)
        out_ref[:, hp * HD:(hp + 1) * HD] = jnp.dot(
            cp, cvm, preferred_element_type=F32)
        cps = cps + cp

    rr = jax.lax.broadcasted_iota(jnp.int32, (NCMP_PAD, NBLK), 0)
    cc = jax.lax.broadcasted_iota(jnp.int32, (NCMP_PAD, NBLK), 1)
    P = jnp.where((rr // 2 == cc) & (rr < NCMP), 1.0, 0.0).astype(F32)
    folded = jnp.dot(cps, P, preferred_element_type=F32)
    impq_ref[0] = jnp.sum(folded.reshape(QC3 // L, L, NBLK), axis=1)


def _k3(q, ck, cv):
    return pl.pallas_call(
        _k3_body,
        grid=(HKV, S // QC3),
        compiler_params=pltpu.CompilerParams(dimension_semantics=("parallel", "parallel")),
        in_specs=[
            pl.BlockSpec((QC3, HPG * HD), lambda g, i: (i, g)),
            pl.BlockSpec((1, NCMP_PAD, HD), lambda g, i: (g, 0, 0)),
            pl.BlockSpec((1, NCMP_PAD, HD), lambda g, i: (g, 0, 0)),
        ],
        out_specs=[
            pl.BlockSpec((QC3, HPG * HD), lambda g, i: (i, g)),
            pl.BlockSpec((1, QC3 // L, NBLK), lambda g, i: (g, i, 0)),
        ],
        out_shape=[
            jax.ShapeDtypeStruct((S, D), F32),
            jax.ShapeDtypeStruct((HKV, NBLK, NBLK), F32),
        ],
    )(q, ck, cv)


def _k4_body(impq_ref, idx_ref):
    vals = impq_ref[0]
    qb = jax.lax.broadcasted_iota(jnp.int32, (NBLK, NBLK), 0)
    mb = jax.lax.broadcasted_iota(jnp.int32, (NBLK, NBLK), 1)
    bonus = jnp.where((mb == qb) | (mb == 0), 1e6, 0.0).astype(F32)
    vals = jnp.where(qb >= mb, vals + bonus, -1e9)

    tcol = jax.lax.broadcasted_iota(jnp.int32, (NBLK, TOPN), 1)
    out = jnp.zeros((NBLK, TOPN), jnp.int32)
    for t in range(TOPN):
        m = jnp.argmax(vals, axis=1).astype(jnp.int32)
        out = jnp.where(tcol == t, m[:, None], out)
        vals = jnp.where(mb == m[:, None], -3e9, vals)
    idx_ref[0] = out


def _k4(impq):
    return pl.pallas_call(
        _k4_body,
        grid=(HKV,),
        compiler_params=pltpu.CompilerParams(dimension_semantics=("parallel",)),
        in_specs=[pl.BlockSpec((1, NBLK, NBLK), lambda g: (g, 0, 0))],
        out_specs=pl.BlockSpec((1, NBLK, TOPN), lambda g: (g, 0, 0)),
        out_shape=jax.ShapeDtypeStruct((HKV, NBLK, TOPN), jnp.int32),
    )(impq)


def _k5_body(idx_ref, q_ref, k_ref, v_ref, out_ref, ks_ref, vs_ref):
    g = pl.program_id(0)
    qb = pl.program_id(1)
    base = g * NBLK * TOPN + qb * TOPN

    rows = jax.lax.broadcasted_iota(jnp.int32, (L, 1), 0)
    qpos = qb * L + rows
    jcol = jax.lax.broadcasted_iota(jnp.int32, (1, TOPN * L), 1)

    colpos = jcol % L
    for t in range(TOPN):
        it = idx_ref[base + t]
        ks_ref[t * L:(t + 1) * L, :] = k_ref[0, pl.ds(it * L, L), :]
        vs_ref[t * L:(t + 1) * L, :] = v_ref[0, pl.ds(it * L, L), :]
        colpos = colpos + jnp.where(jcol // L == t, it * L, 0)
    mask = colpos <= qpos

    ks = ks_ref[:]
    vs = vs_ref[:]
    for hp in range(HPG):
        qh = q_ref[:, hp * HD:(hp + 1) * HD]
        s = jax.lax.dot_general(qh, ks, (((1,), (1,)), ((), ())),
                                preferred_element_type=F32) * SCALE
        s = jnp.where(mask, s, -1e9)
        m = jnp.max(s, axis=-1, keepdims=True)
        p = jnp.exp(s - m)
        p = p / jnp.sum(p, axis=-1, keepdims=True)
        out_ref[:, hp * HD:(hp + 1) * HD] = jnp.dot(
            p, vs, preferred_element_type=F32)


def _k5(top_idx_flat, q, kh, vh):
    grid_spec = pltpu.PrefetchScalarGridSpec(
        num_scalar_prefetch=1,
        grid=(HKV, NBLK),
        in_specs=[
            pl.BlockSpec((L, HPG * HD), lambda g, qb, *_: (qb, g)),
            pl.BlockSpec((1, S, HD), lambda g, qb, *_: (g, 0, 0)),
            pl.BlockSpec((1, S, HD), lambda g, qb, *_: (g, 0, 0)),
        ],
        out_specs=pl.BlockSpec((L, HPG * HD), lambda g, qb, *_: (qb, g)),
        scratch_shapes=[
            pltpu.VMEM((TOPN * L, HD), F32),
            pltpu.VMEM((TOPN * L, HD), F32),
        ],
    )
    return pl.pallas_call(
        _k5_body,
        grid_spec=grid_spec,
        compiler_params=pltpu.CompilerParams(dimension_semantics=("parallel", "parallel")),
        out_shape=jax.ShapeDtypeStruct((S, D), F32),
    )(top_idx_flat, q, kh, vh)


QC6 = 512


def _k6_body(q_ref, kp_ref, kc_ref, vp_ref, vc_ref, out_ref):
    i = pl.program_id(1)
    qpos = i * QC6 + jax.lax.broadcasted_iota(jnp.int32, (QC6, 1), 0)
    col = jax.lax.broadcasted_iota(jnp.int32, (1, 2 * QC6), 1)
    kpos = (i - 1) * QC6 + col
    mask = (qpos >= kpos) & (qpos - kpos < WIN) & ((col >= QC6) | (i > 0))

    kp = kp_ref[0]
    kc = kc_ref[0]
    vcat = jnp.concatenate([vp_ref[0], vc_ref[0]], axis=0)
    for hp in range(HPG):
        qh = q_ref[:, hp * HD:(hp + 1) * HD]
        sp = jax.lax.dot_general(qh, kp, (((1,), (1,)), ((), ())),
                                 preferred_element_type=F32)
        sc = jax.lax.dot_general(qh, kc, (((1,), (1,)), ((), ())),
                                 preferred_element_type=F32)
        s = jnp.concatenate([sp, sc], axis=1) * SCALE
        s = jnp.where(mask, s, -1e9)
        m = jnp.max(s, axis=-1, keepdims=True)
        p = jnp.exp(s - m)
        p = p / jnp.sum(p, axis=-1, keepdims=True)
        out_ref[:, hp * HD:(hp + 1) * HD] = jnp.dot(
            p, vcat, preferred_element_type=F32)


def _k6(q, kh, vh):
    return pl.pallas_call(
        _k6_body,
        grid=(HKV, S // QC6),
        compiler_params=pltpu.CompilerParams(dimension_semantics=("parallel", "parallel")),
        in_specs=[
            pl.BlockSpec((QC6, HPG * HD), lambda g, i: (i, g)),
            pl.BlockSpec((1, QC6, HD), lambda g, i: (g, jnp.maximum(i - 1, 0), 0)),
            pl.BlockSpec((1, QC6, HD), lambda g, i: (g, i, 0)),
            pl.BlockSpec((1, QC6, HD), lambda g, i: (g, jnp.maximum(i - 1, 0), 0)),
            pl.BlockSpec((1, QC6, HD), lambda g, i: (g, i, 0)),
        ],
        out_specs=pl.BlockSpec((QC6, HPG * HD), lambda g, i: (i, g)),
        out_shape=jax.ShapeDtypeStruct((S, D), F32),
    )(q, kh, kh, vh, vh)


def _k7_body(x_ref, cmp_ref, sel_ref, win_ref, g_ref, wo_ref, out_ref):
    gts = g_ref[:]
    rr = jax.lax.broadcasted_iota(jnp.int32, (128, D), 0)
    cc = jax.lax.broadcasted_iota(jnp.int32, (128, D), 1)
    head3 = 3 * (cc // HD)
    e0 = jnp.where(rr == head3, 1.0, 0.0).astype(F32)
    e1 = jnp.where(rr == head3 + 1, 1.0, 0.0).astype(F32)
    e2 = jnp.where(rr == head3 + 2, 1.0, 0.0).astype(F32)
    comb = (cmp_ref[:] * jnp.dot(gts, e0, preferred_element_type=F32)
            + sel_ref[:] * jnp.dot(gts, e1, preferred_element_type=F32)
            + win_ref[:] * jnp.dot(gts, e2, preferred_element_type=F32))
    out_ref[:] = x_ref[:] + jnp.dot(comb, wo_ref[:], preferred_element_type=F32)


def _k7(x, out_cmp, out_sel, out_win, gates, Wo):
    blk = 256
    return pl.pallas_call(
        _k7_body,
        grid=(S // blk,),
        compiler_params=pltpu.CompilerParams(dimension_semantics=("parallel",)),
        in_specs=[
            pl.BlockSpec((blk, D), lambda i: (i, 0)),
            pl.BlockSpec((blk, D), lambda i: (i, 0)),
            pl.BlockSpec((blk, D), lambda i: (i, 0)),
            pl.BlockSpec((blk, D), lambda i: (i, 0)),
            pl.BlockSpec((blk, 128), lambda i: (i, 0)),
            pl.BlockSpec((D, D), lambda i: (0, 0)),
        ],
        out_specs=pl.BlockSpec((blk, D), lambda i: (i, 0)),
        out_shape=jax.ShapeDtypeStruct((S, D), F32),
    )(x, out_cmp, out_sel, out_win, gates, Wo)


def _k8_body(x_ref, g_ref, b_ref, w1_ref, b1_ref, w2_ref, b2_ref, out_ref):
    xb = x_ref[:]
    ln = _ln(xb, g_ref[:], b_ref[:])
    h = jax.nn.gelu(jnp.dot(ln, w1_ref[:], preferred_element_type=F32) + b1_ref[:])
    out_ref[:] = xb + jnp.dot(h, w2_ref[:], preferred_element_type=F32) + b2_ref[:]


def _k8(x1, ln2_g, ln2_b, W1, b1, W2, b2):
    blk = 256
    return pl.pallas_call(
        _k8_body,
        grid=(S // blk,),
        compiler_params=pltpu.CompilerParams(dimension_semantics=("parallel",)),
        in_specs=[
            pl.BlockSpec((blk, D), lambda i: (i, 0)),
            pl.BlockSpec((1, D), lambda i: (0, 0)),
            pl.BlockSpec((1, D), lambda i: (0, 0)),
            pl.BlockSpec((D, 4 * D), lambda i: (0, 0)),
            pl.BlockSpec((1, 4 * D), lambda i: (0, 0)),
            pl.BlockSpec((4 * D, D), lambda i: (0, 0)),
            pl.BlockSpec((1, D), lambda i: (0, 0)),
        ],
        out_specs=pl.BlockSpec((blk, D), lambda i: (i, 0)),
        out_shape=jax.ShapeDtypeStruct((S, D), F32),
    )(x1, ln2_g, ln2_b, W1, b1, W2, b2)


@jax.jit
def _run(x, ln1_g, ln1_b, Wq, Wk, Wv, Wck, bck, Wcv, bcv, Wg, bg, Wo,
         ln2_g, ln2_b, W1, b1, W2, b2):
    x2d = x[0]
    Wg_pad = jnp.pad(Wg, ((0, 0), (0, 128 - 3 * H)))
    bcat = jnp.concatenate(
        [jnp.zeros((D + 2 * HKV * HD,), F32), bg,
         jnp.zeros((128 - 3 * H,), F32)])[None]
    Wcat = jnp.concatenate([Wq, Wk, Wv, Wg_pad], axis=1)

    q, k, v, gates = _k1(x2d, ln1_g[None], ln1_b[None], Wcat, bcat)

    kh = k.reshape(S, HKV, HD).transpose(1, 0, 2)
    vh = v.reshape(S, HKV, HD).transpose(1, 0, 2)
    kf = kh.reshape(HKV, S // STRIDE, STRIDE * HD)
    vf = vh.reshape(HKV, S // STRIDE, STRIDE * HD)

    ck, cv = _k2(kf, vf, Wck, bck[None], Wcv, bcv[None])
    out_cmp, impq = _k3(q, ck, cv)
    top_idx = _k4(impq)
    out_sel = _k5(top_idx.reshape(-1), q, kh, vh)
    out_win = _k6(q, kh, vh)
    x1 = _k7(x2d, out_cmp, out_sel, out_win, gates, Wo)
    out = _k8(x1, ln2_g[None], ln2_b[None], W1, b1[None], W2, b2[None])
    return out[None]


def kernel(x, ln1_g, ln1_b, Wq, Wk, Wv, Wck, bck, Wcv, bcv, Wg, bg, Wo,
           ln2_g, ln2_b, W1, b1, W2, b2):
    return _run(x, ln1_g, ln1_b, Wq, Wk, Wv, Wck, bck, Wcv, bcv, Wg, bg, Wo,
                ln2_g, ln2_b, W1, b1, W2, b2)


# speedup vs baseline: 1.5157x; 1.5157x over previous
import functools

import jax
import jax.numpy as jnp
import numpy as np
from jax.experimental import pallas as pl
from jax.experimental.pallas import tpu as pltpu

D = 768
H = 12
HKV = 3
HPG = H // HKV
HD = 64
L = 32
STRIDE = 16
TOPN = 16
WIN = 512
S = 2048
NCMP = (S - L) // STRIDE + 1
NCMP_PAD = 128
NBLK = S // L
SCALE = 1.0 / np.sqrt(HD)

F32 = jnp.float32


def _ln(xb, g, b):
    m = jnp.mean(xb, axis=-1, keepdims=True)
    v = jnp.var(xb, axis=-1, keepdims=True)
    return (xb - m) * jax.lax.rsqrt(v + 1e-5) * g + b


def _k1_body(x_ref, g_ref, b_ref, w_ref, bc_ref, q_ref, k_ref, v_ref, gt_ref):
    xb = x_ref[:]
    ln = _ln(xb, g_ref[:], b_ref[:])
    out = jnp.dot(ln, w_ref[:], preferred_element_type=F32) + bc_ref[:]
    q_ref[:] = out[:, :D]
    k_ref[:] = out[:, D:D + HKV * HD]
    v_ref[:] = out[:, D + HKV * HD:D + 2 * HKV * HD]
    gt_ref[:] = jax.nn.sigmoid(out[:, D + 2 * HKV * HD:])


def _k1(x, ln1_g, ln1_b, Wcat, bcat):
    blk = 256
    return pl.pallas_call(
        _k1_body,
        grid=(S // blk,),
        compiler_params=pltpu.CompilerParams(dimension_semantics=("parallel",)),
        in_specs=[
            pl.BlockSpec((blk, D), lambda i: (i, 0)),
            pl.BlockSpec((1, D), lambda i: (0, 0)),
            pl.BlockSpec((1, D), lambda i: (0, 0)),
            pl.BlockSpec(Wcat.shape, lambda i: (0, 0)),
            pl.BlockSpec((1, Wcat.shape[1]), lambda i: (0, 0)),
        ],
        out_specs=[
            pl.BlockSpec((blk, D), lambda i: (i, 0)),
            pl.BlockSpec((blk, HKV * HD), lambda i: (i, 0)),
            pl.BlockSpec((blk, HKV * HD), lambda i: (i, 0)),
            pl.BlockSpec((blk, 128), lambda i: (i, 0)),
        ],
        out_shape=[
            jax.ShapeDtypeStruct((S, D), F32),
            jax.ShapeDtypeStruct((S, HKV * HD), F32),
            jax.ShapeDtypeStruct((S, HKV * HD), F32),
            jax.ShapeDtypeStruct((S, 128), F32),
        ],
    )(x, ln1_g, ln1_b, Wcat, bcat)


def _k2_body(kf_ref, vf_ref, wk_ref, bk_ref, wv_ref, bv_ref, ck_ref, cv_ref):
    kr = kf_ref[0]
    vr = vf_ref[0]
    zero = jnp.zeros((1, HD), F32)

    def proj(r, w_ref, b_ref):
        t0 = jnp.dot(r, w_ref[:STRIDE * HD], preferred_element_type=F32)
        t1 = jnp.dot(r, w_ref[STRIDE * HD:], preferred_element_type=F32)
        t1s = jnp.concatenate([t1[1:], zero], axis=0)
        return t0 + t1s + b_ref[:]

    ck_ref[0] = proj(kr, wk_ref, bk_ref)
    cv_ref[0] = proj(vr, wv_ref, bv_ref)


def _k2(kflat, vflat, Wck, bck, Wcv, bcv):
    return pl.pallas_call(
        _k2_body,
        grid=(HKV,),
        compiler_params=pltpu.CompilerParams(dimension_semantics=("parallel",)),
        in_specs=[
            pl.BlockSpec((1, S // STRIDE, STRIDE * HD), lambda g: (g, 0, 0)),
            pl.BlockSpec((1, S // STRIDE, STRIDE * HD), lambda g: (g, 0, 0)),
            pl.BlockSpec(Wck.shape, lambda g: (0, 0)),
            pl.BlockSpec((1, HD), lambda g: (0, 0)),
            pl.BlockSpec(Wcv.shape, lambda g: (0, 0)),
            pl.BlockSpec((1, HD), lambda g: (0, 0)),
        ],
        out_specs=[
            pl.BlockSpec((1, NCMP_PAD, HD), lambda g: (g, 0, 0)),
            pl.BlockSpec((1, NCMP_PAD, HD), lambda g: (g, 0, 0)),
        ],
        out_shape=[
            jax.ShapeDtypeStruct((HKV, NCMP_PAD, HD), F32),
            jax.ShapeDtypeStruct((HKV, NCMP_PAD, HD), F32),
        ],
    )(kflat, vflat, Wck, bck, Wcv, bcv)


QC3 = 512


def _k3_body(q_ref, ck_ref, cv_ref, out_ref, impq_ref):
    i = pl.program_id(1)
    ckm = ck_ref[0]
    cvm = cv_ref[0]
    qpos = i * QC3 + jax.lax.broadcasted_iota(jnp.int32, (QC3, 1), 0)
    nidx = jax.lax.broadcasted_iota(jnp.int32, (1, NCMP_PAD), 1)
    cmp_end = nidx * STRIDE + (L - 1)
    mask = qpos >= cmp_end
    pad = nidx < NCMP

    cps = jnp.zeros((QC3, NCMP_PAD), F32)
    for hp in range(HPG):
        qh = q_ref[:, hp * HD:(hp + 1) * HD]
        s = jax.lax.dot_general(qh, ckm, (((1,), (1,)), ((), ())),
                                preferred_element_type=F32) * SCALE
        s = jnp.where(mask, s, -1e9)
        s = jnp.where(pad, s, -1e30)
        m = jnp.max(s, axis=-1, keepdims=True)
        p = jnp.exp(s - m)
        cp = p / jnp.sum(p, axis=-1, keepdims=True)
        out_ref[:, hp * HD:(hp + 1) * HD] = jnp.dot(
            cp, cvm, preferred_element_type=F32)
        cps = cps + cp

    rr = jax.lax.broadcasted_iota(jnp.int32, (NCMP_PAD, NBLK), 0)
    cc = jax.lax.broadcasted_iota(jnp.int32, (NCMP_PAD, NBLK), 1)
    P = jnp.where((rr // 2 == cc) & (rr < NCMP), 1.0, 0.0).astype(F32)
    folded = jnp.dot(cps, P, preferred_element_type=F32)
    impq_ref[0] = jnp.sum(folded.reshape(QC3 // L, L, NBLK), axis=1)


def _k3(q, ck, cv):
    return pl.pallas_call(
        _k3_body,
        grid=(HKV, S // QC3),
        compiler_params=pltpu.CompilerParams(dimension_semantics=("parallel", "parallel")),
        in_specs=[
            pl.BlockSpec((QC3, HPG * HD), lambda g, i: (i, g)),
            pl.BlockSpec((1, NCMP_PAD, HD), lambda g, i: (g, 0, 0)),
            pl.BlockSpec((1, NCMP_PAD, HD), lambda g, i: (g, 0, 0)),
        ],
        out_specs=[
            pl.BlockSpec((QC3, HPG * HD), lambda g, i: (i, g)),
            pl.BlockSpec((1, QC3 // L, NBLK), lambda g, i: (g, i, 0)),
        ],
        out_shape=[
            jax.ShapeDtypeStruct((S, D), F32),
            jax.ShapeDtypeStruct((HKV, NBLK, NBLK), F32),
        ],
    )(q, ck, cv)


def _k4_body(impq_ref, idx_ref):
    vals = impq_ref[0]
    qb = jax.lax.broadcasted_iota(jnp.int32, (NBLK, NBLK), 0)
    mb = jax.lax.broadcasted_iota(jnp.int32, (NBLK, NBLK), 1)
    bonus = jnp.where((mb == qb) | (mb == 0), 1e6, 0.0).astype(F32)
    vals = jnp.where(qb >= mb, vals + bonus, -1e9)

    tcol = jax.lax.broadcasted_iota(jnp.int32, (NBLK, TOPN), 1)
    out = jnp.zeros((NBLK, TOPN), jnp.int32)
    for t in range(TOPN):
        m = jnp.argmax(vals, axis=1).astype(jnp.int32)
        out = jnp.where(tcol == t, m[:, None], out)
        vals = jnp.where(mb == m[:, None], -3e9, vals)
    idx_ref[0] = out


def _k4(impq):
    return pl.pallas_call(
        _k4_body,
        grid=(HKV,),
        compiler_params=pltpu.CompilerParams(dimension_semantics=("parallel",)),
        in_specs=[pl.BlockSpec((1, NBLK, NBLK), lambda g: (g, 0, 0))],
        out_specs=pl.BlockSpec((1, NBLK, TOPN), lambda g: (g, 0, 0)),
        out_shape=jax.ShapeDtypeStruct((HKV, NBLK, TOPN), jnp.int32),
    )(impq)


def _k5_body(idx_ref, q_ref, kv_ref, out_ref, kv_scr):
    g = pl.program_id(0)
    qb = pl.program_id(1)
    base = g * NBLK * TOPN + qb * TOPN

    rows = jax.lax.broadcasted_iota(jnp.int32, (HPG * L, 1), 0)
    qpos = qb * L + rows % L
    jcol = jax.lax.broadcasted_iota(jnp.int32, (1, TOPN * L), 1)

    colpos = jcol % L
    for t in range(TOPN):
        it = idx_ref[base + t]
        kv_scr[t * L:(t + 1) * L, :] = kv_ref[0, pl.ds(it * L, L), :]
        colpos = colpos + jnp.where(jcol // L == t, it * L, 0)
    mask = colpos <= qpos

    ks = kv_scr[:, :HD]
    vs = kv_scr[:, HD:]
    q4 = jnp.concatenate(
        [q_ref[:, hp * HD:(hp + 1) * HD] for hp in range(HPG)], axis=0)
    s = jax.lax.dot_general(q4, ks, (((1,), (1,)), ((), ())),
                            preferred_element_type=F32) * SCALE
    s = jnp.where(mask, s, -1e9)
    m = jnp.max(s, axis=-1, keepdims=True)
    p = jnp.exp(s - m)
    r = 1.0 / jnp.sum(p, axis=-1, keepdims=True)
    o = jnp.dot(p, vs, preferred_element_type=F32) * r
    for hp in range(HPG):
        out_ref[:, hp * HD:(hp + 1) * HD] = o[hp * L:(hp + 1) * L, :]


def _k5(top_idx_flat, q, kvh):
    grid_spec = pltpu.PrefetchScalarGridSpec(
        num_scalar_prefetch=1,
        grid=(HKV, NBLK),
        in_specs=[
            pl.BlockSpec((L, HPG * HD), lambda g, qb, *_: (qb, g)),
            pl.BlockSpec((1, S, 2 * HD), lambda g, qb, *_: (g, 0, 0)),
        ],
        out_specs=pl.BlockSpec((L, HPG * HD), lambda g, qb, *_: (qb, g)),
        scratch_shapes=[
            pltpu.VMEM((TOPN * L, 2 * HD), F32),
        ],
    )
    return pl.pallas_call(
        _k5_body,
        grid_spec=grid_spec,
        compiler_params=pltpu.CompilerParams(dimension_semantics=("parallel", "parallel")),
        out_shape=jax.ShapeDtypeStruct((S, D), F32),
    )(top_idx_flat, q, kvh)


QC6 = 256
NV6 = WIN // QC6 + 1


def _k6_body(q_ref, kv0_ref, kv1_ref, kv2_ref, out_ref):
    i = pl.program_id(1)
    qpos = i * QC6 + jax.lax.broadcasted_iota(jnp.int32, (QC6, 1), 0)
    col = jax.lax.broadcasted_iota(jnp.int32, (1, NV6 * QC6), 1)
    kpos = (i - (NV6 - 1)) * QC6 + col
    mask = (qpos >= kpos) & (qpos - kpos < WIN) & (col // QC6 >= (NV6 - 1) - i)

    kv = jnp.concatenate([kv0_ref[0], kv1_ref[0], kv2_ref[0]], axis=0)
    kcat = kv[:, :HD]
    vcat = kv[:, HD:]
    for hp in range(HPG):
        qh = q_ref[:, hp * HD:(hp + 1) * HD]
        s = jax.lax.dot_general(qh, kcat, (((1,), (1,)), ((), ())),
                                preferred_element_type=F32) * SCALE
        s = jnp.where(mask, s, -1e9)
        m = jnp.max(s, axis=-1, keepdims=True)
        p = jnp.exp(s - m)
        r = 1.0 / jnp.sum(p, axis=-1, keepdims=True)
        out_ref[:, hp * HD:(hp + 1) * HD] = jnp.dot(
            p, vcat, preferred_element_type=F32) * r


def _k6(q, kvh):
    kvspec = lambda d: pl.BlockSpec(
        (1, QC6, 2 * HD), lambda g, i, d=d: (g, jnp.maximum(i - d, 0), 0))
    return pl.pallas_call(
        _k6_body,
        grid=(HKV, S // QC6),
        compiler_params=pltpu.CompilerParams(dimension_semantics=("parallel", "parallel")),
        in_specs=[
            pl.BlockSpec((QC6, HPG * HD), lambda g, i: (i, g)),
            kvspec(2),
            kvspec(1),
            kvspec(0),
        ],
        out_specs=pl.BlockSpec((QC6, HPG * HD), lambda g, i: (i, g)),
        out_shape=jax.ShapeDtypeStruct((S, D), F32),
    )(q, kvh, kvh, kvh)


def _k7_body(x_ref, cmp_ref, sel_ref, win_ref, g_ref, wo_ref,
             ln2g_ref, ln2b_ref, w1_ref, b1_ref, w2_ref, b2_ref, out_ref):
    gts = g_ref[:]
    rr = jax.lax.broadcasted_iota(jnp.int32, (128, D), 0)
    cc = jax.lax.broadcasted_iota(jnp.int32, (128, D), 1)
    head3 = 3 * (cc // HD)
    e0 = jnp.where(rr == head3, 1.0, 0.0).astype(F32)
    e1 = jnp.where(rr == head3 + 1, 1.0, 0.0).astype(F32)
    e2 = jnp.where(rr == head3 + 2, 1.0, 0.0).astype(F32)
    comb = (cmp_ref[:] * jnp.dot(gts, e0, preferred_element_type=F32)
            + sel_ref[:] * jnp.dot(gts, e1, preferred_element_type=F32)
            + win_ref[:] * jnp.dot(gts, e2, preferred_element_type=F32))
    x1 = x_ref[:] + jnp.dot(comb, wo_ref[:], preferred_element_type=F32)
    ln = _ln(x1, ln2g_ref[:], ln2b_ref[:])
    h = jax.nn.gelu(jnp.dot(ln, w1_ref[:], preferred_element_type=F32) + b1_ref[:])
    out_ref[:] = x1 + jnp.dot(h, w2_ref[:], preferred_element_type=F32) + b2_ref[:]


def _k7(x, out_cmp, out_sel, out_win, gates, Wo, ln2_g, ln2_b, W1, b1, W2, b2):
    blk = 256
    return pl.pallas_call(
        _k7_body,
        grid=(S // blk,),
        compiler_params=pltpu.CompilerParams(dimension_semantics=("parallel",)),
        in_specs=[
            pl.BlockSpec((blk, D), lambda i: (i, 0)),
            pl.BlockSpec((blk, D), lambda i: (i, 0)),
            pl.BlockSpec((blk, D), lambda i: (i, 0)),
            pl.BlockSpec((blk, D), lambda i: (i, 0)),
            pl.BlockSpec((blk, 128), lambda i: (i, 0)),
            pl.BlockSpec((D, D), lambda i: (0, 0)),
            pl.BlockSpec((1, D), lambda i: (0, 0)),
            pl.BlockSpec((1, D), lambda i: (0, 0)),
            pl.BlockSpec((D, 4 * D), lambda i: (0, 0)),
            pl.BlockSpec((1, 4 * D), lambda i: (0, 0)),
            pl.BlockSpec((4 * D, D), lambda i: (0, 0)),
            pl.BlockSpec((1, D), lambda i: (0, 0)),
        ],
        out_specs=pl.BlockSpec((blk, D), lambda i: (i, 0)),
        out_shape=jax.ShapeDtypeStruct((S, D), F32),
    )(x, out_cmp, out_sel, out_win, gates, Wo, ln2_g, ln2_b, W1, b1, W2, b2)


@jax.jit
def _run(x, ln1_g, ln1_b, Wq, Wk, Wv, Wck, bck, Wcv, bcv, Wg, bg, Wo,
         ln2_g, ln2_b, W1, b1, W2, b2):
    x2d = x[0]
    Wg_pad = jnp.pad(Wg, ((0, 0), (0, 128 - 3 * H)))
    bcat = jnp.concatenate(
        [jnp.zeros((D + 2 * HKV * HD,), F32), bg,
         jnp.zeros((128 - 3 * H,), F32)])[None]
    Wcat = jnp.concatenate([Wq, Wk, Wv, Wg_pad], axis=1)

    q, k, v, gates = _k1(x2d, ln1_g[None], ln1_b[None], Wcat, bcat)

    kh = k.reshape(S, HKV, HD).transpose(1, 0, 2)
    vh = v.reshape(S, HKV, HD).transpose(1, 0, 2)
    kvh = jnp.concatenate([kh, vh], axis=-1)
    kf = kh.reshape(HKV, S // STRIDE, STRIDE * HD)
    vf = vh.reshape(HKV, S // STRIDE, STRIDE * HD)

    ck, cv = _k2(kf, vf, Wck, bck[None], Wcv, bcv[None])
    out_cmp, impq = _k3(q, ck, cv)
    top_idx = _k4(impq)
    out_sel = _k5(top_idx.reshape(-1), q, kvh)
    out_win = _k6(q, kvh)
    out = _k7(x2d, out_cmp, out_sel, out_win, gates, Wo,
              ln2_g[None], ln2_b[None], W1, b1[None], W2, b2[None])
    return out[None]


def kernel(x, ln1_g, ln1_b, Wq, Wk, Wv, Wck, bck, Wcv, bcv, Wg, bg, Wo,
           ln2_g, ln2_b, W1, b1, W2, b2):
    return _run(x, ln1_g, ln1_b, Wq, Wk, Wv, Wck, bck, Wcv, bcv, Wg, bg, Wo,
                ln2_g, ln2_b, W1, b1, W2, b2)


# speedup vs baseline: 1.9881x; 1.3117x over previous
import functools

import jax
import jax.numpy as jnp
import numpy as np
from jax.experimental import pallas as pl
from jax.experimental.pallas import tpu as pltpu

D = 768
H = 12
HKV = 3
HPG = H // HKV
HD = 64
L = 32
STRIDE = 16
TOPN = 16
WIN = 512
S = 2048
NCMP = (S - L) // STRIDE + 1
NCMP_PAD = 128
NBLK = S // L
SCALE = 1.0 / np.sqrt(HD)

F32 = jnp.float32


def _ln(xb, g, b):
    m = jnp.mean(xb, axis=-1, keepdims=True)
    v = jnp.var(xb, axis=-1, keepdims=True)
    return (xb - m) * jax.lax.rsqrt(v + 1e-5) * g + b


def _k1_body(x_ref, g_ref, b_ref, w_ref, bc_ref, q_ref, k_ref, v_ref, gt_ref):
    xb = x_ref[:]
    ln = _ln(xb, g_ref[:], b_ref[:])
    out = jnp.dot(ln, w_ref[:], preferred_element_type=F32) + bc_ref[:]
    q_ref[:] = out[:, :D]
    k_ref[:] = out[:, D:D + HKV * HD]
    v_ref[:] = out[:, D + HKV * HD:D + 2 * HKV * HD]
    gt_ref[:] = jax.nn.sigmoid(out[:, D + 2 * HKV * HD:])


def _k1(x, ln1_g, ln1_b, Wcat, bcat):
    blk = 256
    return pl.pallas_call(
        _k1_body,
        grid=(S // blk,),
        compiler_params=pltpu.CompilerParams(dimension_semantics=("parallel",)),
        in_specs=[
            pl.BlockSpec((blk, D), lambda i: (i, 0)),
            pl.BlockSpec((1, D), lambda i: (0, 0)),
            pl.BlockSpec((1, D), lambda i: (0, 0)),
            pl.BlockSpec(Wcat.shape, lambda i: (0, 0)),
            pl.BlockSpec((1, Wcat.shape[1]), lambda i: (0, 0)),
        ],
        out_specs=[
            pl.BlockSpec((blk, D), lambda i: (i, 0)),
            pl.BlockSpec((blk, HKV * HD), lambda i: (i, 0)),
            pl.BlockSpec((blk, HKV * HD), lambda i: (i, 0)),
            pl.BlockSpec((blk, 128), lambda i: (i, 0)),
        ],
        out_shape=[
            jax.ShapeDtypeStruct((S, D), F32),
            jax.ShapeDtypeStruct((S, HKV * HD), F32),
            jax.ShapeDtypeStruct((S, HKV * HD), F32),
            jax.ShapeDtypeStruct((S, 128), F32),
        ],
    )(x, ln1_g, ln1_b, Wcat, bcat)


def _k2_body(kf_ref, vf_ref, wk_ref, bk_ref, wv_ref, bv_ref, ck_ref, cv_ref):
    kr = kf_ref[0]
    vr = vf_ref[0]
    zero = jnp.zeros((1, HD), F32)

    def proj(r, w_ref, b_ref):
        t0 = jnp.dot(r, w_ref[:STRIDE * HD], preferred_element_type=F32)
        t1 = jnp.dot(r, w_ref[STRIDE * HD:], preferred_element_type=F32)
        t1s = jnp.concatenate([t1[1:], zero], axis=0)
        return t0 + t1s + b_ref[:]

    ck_ref[0] = proj(kr, wk_ref, bk_ref)
    cv_ref[0] = proj(vr, wv_ref, bv_ref)


def _k2(kflat, vflat, Wck, bck, Wcv, bcv):
    return pl.pallas_call(
        _k2_body,
        grid=(HKV,),
        compiler_params=pltpu.CompilerParams(dimension_semantics=("parallel",)),
        in_specs=[
            pl.BlockSpec((1, S // STRIDE, STRIDE * HD), lambda g: (g, 0, 0)),
            pl.BlockSpec((1, S // STRIDE, STRIDE * HD), lambda g: (g, 0, 0)),
            pl.BlockSpec(Wck.shape, lambda g: (0, 0)),
            pl.BlockSpec((1, HD), lambda g: (0, 0)),
            pl.BlockSpec(Wcv.shape, lambda g: (0, 0)),
            pl.BlockSpec((1, HD), lambda g: (0, 0)),
        ],
        out_specs=[
            pl.BlockSpec((1, NCMP_PAD, HD), lambda g: (g, 0, 0)),
            pl.BlockSpec((1, NCMP_PAD, HD), lambda g: (g, 0, 0)),
        ],
        out_shape=[
            jax.ShapeDtypeStruct((HKV, NCMP_PAD, HD), F32),
            jax.ShapeDtypeStruct((HKV, NCMP_PAD, HD), F32),
        ],
    )(kflat, vflat, Wck, bck, Wcv, bcv)


QC3 = 512


def _k3_body(q_ref, ck_ref, cv_ref, out_ref, impq_ref):
    i = pl.program_id(1)
    ckm = ck_ref[0]
    cvm = cv_ref[0]
    qpos = i * QC3 + jax.lax.broadcasted_iota(jnp.int32, (QC3, 1), 0)
    nidx = jax.lax.broadcasted_iota(jnp.int32, (1, NCMP_PAD), 1)
    cmp_end = nidx * STRIDE + (L - 1)
    mask = qpos >= cmp_end
    pad = nidx < NCMP

    cps = jnp.zeros((QC3, NCMP_PAD), F32)
    for hp in range(HPG):
        qh = q_ref[:, hp * HD:(hp + 1) * HD]
        s = jax.lax.dot_general(qh, ckm, (((1,), (1,)), ((), ())),
                                preferred_element_type=F32) * SCALE
        s = jnp.where(mask, s, -1e9)
        s = jnp.where(pad, s, -1e30)
        m = jnp.max(s, axis=-1, keepdims=True)
        p = jnp.exp(s - m)
        cp = p / jnp.sum(p, axis=-1, keepdims=True)
        out_ref[:, hp * HD:(hp + 1) * HD] = jnp.dot(
            cp, cvm, preferred_element_type=F32)
        cps = cps + cp

    rr = jax.lax.broadcasted_iota(jnp.int32, (NCMP_PAD, NBLK), 0)
    cc = jax.lax.broadcasted_iota(jnp.int32, (NCMP_PAD, NBLK), 1)
    P = jnp.where((rr // 2 == cc) & (rr < NCMP), 1.0, 0.0).astype(F32)
    folded = jnp.dot(cps, P, preferred_element_type=F32)
    impq_ref[0] = jnp.sum(folded.reshape(QC3 // L, L, NBLK), axis=1)


def _k3(q, ck, cv):
    return pl.pallas_call(
        _k3_body,
        grid=(HKV, S // QC3),
        compiler_params=pltpu.CompilerParams(dimension_semantics=("parallel", "parallel")),
        in_specs=[
            pl.BlockSpec((QC3, HPG * HD), lambda g, i: (i, g)),
            pl.BlockSpec((1, NCMP_PAD, HD), lambda g, i: (g, 0, 0)),
            pl.BlockSpec((1, NCMP_PAD, HD), lambda g, i: (g, 0, 0)),
        ],
        out_specs=[
            pl.BlockSpec((QC3, HPG * HD), lambda g, i: (i, g)),
            pl.BlockSpec((1, QC3 // L, NBLK), lambda g, i: (g, i, 0)),
        ],
        out_shape=[
            jax.ShapeDtypeStruct((S, D), F32),
            jax.ShapeDtypeStruct((HKV, NBLK, NBLK), F32),
        ],
    )(q, ck, cv)


def _k4_body(impq_ref, idx_ref):
    vals = impq_ref[:]
    R = HKV * NBLK
    qb = jax.lax.broadcasted_iota(jnp.int32, (R, NBLK), 0) % NBLK
    mb = jax.lax.broadcasted_iota(jnp.int32, (R, NBLK), 1)
    bonus = jnp.where((mb == qb) | (mb == 0), 1e6, 0.0).astype(F32)
    vals = jnp.where(qb >= mb, vals + bonus, -1e9)

    tcol = jax.lax.broadcasted_iota(jnp.int32, (R, TOPN), 1)
    out = jnp.zeros((R, TOPN), jnp.int32)
    for t in range(TOPN):
        m = jnp.argmax(vals, axis=1).astype(jnp.int32)
        out = jnp.where(tcol == t, m[:, None], out)
        vals = jnp.where(mb == m[:, None], -3e9, vals)
    idx_ref[:] = out


def _k4(impq):
    return pl.pallas_call(
        _k4_body,
        out_shape=jax.ShapeDtypeStruct((HKV * NBLK, TOPN), jnp.int32),
    )(impq.reshape(HKV * NBLK, NBLK))


QB5 = 8


def _k5_body(idx_ref, q_ref, kv_ref, out_ref, kv_scr):
    g = pl.program_id(0)
    i = pl.program_id(1)

    rows = jax.lax.broadcasted_iota(jnp.int32, (HPG * L, 1), 0)
    jcol = jax.lax.broadcasted_iota(jnp.int32, (1, TOPN * L), 1)
    jmod = jcol % L

    for qq in range(QB5):
        qb = i * QB5 + qq
        base = g * NBLK * TOPN + qb * TOPN
        qpos = qb * L + rows % L

        colpos = jmod
        for t in range(TOPN):
            it = idx_ref[base + t]
            kv_scr[qq * TOPN * L + t * L:qq * TOPN * L + (t + 1) * L, :] = (
                kv_ref[0, pl.ds(it * L, L), :])
            colpos = colpos + jnp.where(jcol // L == t, it * L, 0)
        mask = colpos <= qpos

        ks = kv_scr[qq * TOPN * L:(qq + 1) * TOPN * L, :HD]
        vs = kv_scr[qq * TOPN * L:(qq + 1) * TOPN * L, HD:]
        q4 = jnp.concatenate(
            [q_ref[qq * L:(qq + 1) * L, hp * HD:(hp + 1) * HD]
             for hp in range(HPG)], axis=0)
        s = jax.lax.dot_general(q4, ks, (((1,), (1,)), ((), ())),
                                preferred_element_type=F32) * SCALE
        s = jnp.where(mask, s, -1e9)
        m = jnp.max(s, axis=-1, keepdims=True)
        p = jnp.exp(s - m)
        r = 1.0 / jnp.sum(p, axis=-1, keepdims=True)
        o = jnp.dot(p, vs, preferred_element_type=F32) * r
        for hp in range(HPG):
            out_ref[qq * L:(qq + 1) * L, hp * HD:(hp + 1) * HD] = (
                o[hp * L:(hp + 1) * L, :])


def _k5(top_idx_flat, q, kvh):
    grid_spec = pltpu.PrefetchScalarGridSpec(
        num_scalar_prefetch=1,
        grid=(HKV, NBLK // QB5),
        in_specs=[
            pl.BlockSpec((QB5 * L, HPG * HD), lambda g, i, *_: (i, g)),
            pl.BlockSpec((1, S, 2 * HD), lambda g, i, *_: (g, 0, 0)),
        ],
        out_specs=pl.BlockSpec((QB5 * L, HPG * HD), lambda g, i, *_: (i, g)),
        scratch_shapes=[
            pltpu.VMEM((QB5 * TOPN * L, 2 * HD), F32),
        ],
    )
    return pl.pallas_call(
        _k5_body,
        grid_spec=grid_spec,
        compiler_params=pltpu.CompilerParams(dimension_semantics=("parallel", "parallel")),
        out_shape=jax.ShapeDtypeStruct((S, D), F32),
    )(top_idx_flat, q, kvh)


QC6 = 256
NV6 = WIN // QC6 + 1


def _k6_body(q_ref, kv0_ref, kv1_ref, kv2_ref, out_ref):
    i = pl.program_id(1)
    qpos = i * QC6 + jax.lax.broadcasted_iota(jnp.int32, (QC6, 1), 0)
    col = jax.lax.broadcasted_iota(jnp.int32, (1, NV6 * QC6), 1)
    kpos = (i - (NV6 - 1)) * QC6 + col
    mask = (qpos >= kpos) & (qpos - kpos < WIN) & (col // QC6 >= (NV6 - 1) - i)

    kv = jnp.concatenate([kv0_ref[0], kv1_ref[0], kv2_ref[0]], axis=0)
    kcat = kv[:, :HD]
    vcat = kv[:, HD:]
    for hp in range(HPG):
        qh = q_ref[:, hp * HD:(hp + 1) * HD]
        s = jax.lax.dot_general(qh, kcat, (((1,), (1,)), ((), ())),
                                preferred_element_type=F32) * SCALE
        s = jnp.where(mask, s, -1e9)
        m = jnp.max(s, axis=-1, keepdims=True)
        p = jnp.exp(s - m)
        r = 1.0 / jnp.sum(p, axis=-1, keepdims=True)
        out_ref[:, hp * HD:(hp + 1) * HD] = jnp.dot(
            p, vcat, preferred_element_type=F32) * r


def _k6(q, kvh):
    kvspec = lambda d: pl.BlockSpec(
        (1, QC6, 2 * HD), lambda g, i, d=d: (g, jnp.maximum(i - d, 0), 0))
    return pl.pallas_call(
        _k6_body,
        grid=(HKV, S // QC6),
        compiler_params=pltpu.CompilerParams(dimension_semantics=("parallel", "parallel")),
        in_specs=[
            pl.BlockSpec((QC6, HPG * HD), lambda g, i: (i, g)),
            kvspec(2),
            kvspec(1),
            kvspec(0),
        ],
        out_specs=pl.BlockSpec((QC6, HPG * HD), lambda g, i: (i, g)),
        out_shape=jax.ShapeDtypeStruct((S, D), F32),
    )(q, kvh, kvh, kvh)


def _k7_body(x_ref, cmp_ref, sel_ref, win_ref, g_ref, wo_ref,
             ln2g_ref, ln2b_ref, w1_ref, b1_ref, w2_ref, b2_ref, out_ref):
    gts = g_ref[:]
    rr = jax.lax.broadcasted_iota(jnp.int32, (128, D), 0)
    cc = jax.lax.broadcasted_iota(jnp.int32, (128, D), 1)
    head3 = 3 * (cc // HD)
    e0 = jnp.where(rr == head3, 1.0, 0.0).astype(F32)
    e1 = jnp.where(rr == head3 + 1, 1.0, 0.0).astype(F32)
    e2 = jnp.where(rr == head3 + 2, 1.0, 0.0).astype(F32)
    comb = (cmp_ref[:] * jnp.dot(gts, e0, preferred_element_type=F32)
            + sel_ref[:] * jnp.dot(gts, e1, preferred_element_type=F32)
            + win_ref[:] * jnp.dot(gts, e2, preferred_element_type=F32))
    x1 = x_ref[:] + jnp.dot(comb, wo_ref[:], preferred_element_type=F32)
    ln = _ln(x1, ln2g_ref[:], ln2b_ref[:])
    h = jax.nn.gelu(jnp.dot(ln, w1_ref[:], preferred_element_type=F32) + b1_ref[:])
    out_ref[:] = x1 + jnp.dot(h, w2_ref[:], preferred_element_type=F32) + b2_ref[:]


def _k7(x, out_cmp, out_sel, out_win, gates, Wo, ln2_g, ln2_b, W1, b1, W2, b2):
    blk = 256
    return pl.pallas_call(
        _k7_body,
        grid=(S // blk,),
        compiler_params=pltpu.CompilerParams(dimension_semantics=("parallel",)),
        in_specs=[
            pl.BlockSpec((blk, D), lambda i: (i, 0)),
            pl.BlockSpec((blk, D), lambda i: (i, 0)),
            pl.BlockSpec((blk, D), lambda i: (i, 0)),
            pl.BlockSpec((blk, D), lambda i: (i, 0)),
            pl.BlockSpec((blk, 128), lambda i: (i, 0)),
            pl.BlockSpec((D, D), lambda i: (0, 0)),
            pl.BlockSpec((1, D), lambda i: (0, 0)),
            pl.BlockSpec((1, D), lambda i: (0, 0)),
            pl.BlockSpec((D, 4 * D), lambda i: (0, 0)),
            pl.BlockSpec((1, 4 * D), lambda i: (0, 0)),
            pl.BlockSpec((4 * D, D), lambda i: (0, 0)),
            pl.BlockSpec((1, D), lambda i: (0, 0)),
        ],
        out_specs=pl.BlockSpec((blk, D), lambda i: (i, 0)),
        out_shape=jax.ShapeDtypeStruct((S, D), F32),
    )(x, out_cmp, out_sel, out_win, gates, Wo, ln2_g, ln2_b, W1, b1, W2, b2)


@jax.jit
def _run(x, ln1_g, ln1_b, Wq, Wk, Wv, Wck, bck, Wcv, bcv, Wg, bg, Wo,
         ln2_g, ln2_b, W1, b1, W2, b2):
    x2d = x[0]
    Wg_pad = jnp.pad(Wg, ((0, 0), (0, 128 - 3 * H)))
    bcat = jnp.concatenate(
        [jnp.zeros((D + 2 * HKV * HD,), F32), bg,
         jnp.zeros((128 - 3 * H,), F32)])[None]
    Wcat = jnp.concatenate([Wq, Wk, Wv, Wg_pad], axis=1)

    q, k, v, gates = _k1(x2d, ln1_g[None], ln1_b[None], Wcat, bcat)

    kh = k.reshape(S, HKV, HD).transpose(1, 0, 2)
    vh = v.reshape(S, HKV, HD).transpose(1, 0, 2)
    kvh = jnp.concatenate([kh, vh], axis=-1)
    kf = kh.reshape(HKV, S // STRIDE, STRIDE * HD)
    vf = vh.reshape(HKV, S // STRIDE, STRIDE * HD)

    ck, cv = _k2(kf, vf, Wck, bck[None], Wcv, bcv[None])
    out_cmp, impq = _k3(q, ck, cv)
    top_idx = _k4(impq)
    out_sel = _k5(top_idx.reshape(-1), q, kvh)
    out_win = _k6(q, kvh)
    out = _k7(x2d, out_cmp, out_sel, out_win, gates, Wo,
              ln2_g[None], ln2_b[None], W1, b1[None], W2, b2[None])
    return out[None]


def kernel(x, ln1_g, ln1_b, Wq, Wk, Wv, Wck, bck, Wcv, bcv, Wg, bg, Wo,
           ln2_g, ln2_b, W1, b1, W2, b2):
    return _run(x, ln1_g, ln1_b, Wq, Wk, Wv, Wck, bck, Wcv, bcv, Wg, bg, Wo,
                ln2_g, ln2_b, W1, b1, W2, b2)


# speedup vs baseline: 2.0654x; 1.0388x over previous
import jax
import jax.numpy as jnp
import numpy as np
from jax.experimental import pallas as pl
from jax.experimental.pallas import tpu as pltpu

D = 768
H = 12
HKV = 3
HPG = H // HKV
HD = 64
L = 32
STRIDE = 16
TOPN = 16
WIN = 512
S = 2048
NCMP = (S - L) // STRIDE + 1
NCMP_PAD = 128
NBLK = S // L
SCALE = 1.0 / np.sqrt(HD)

F32 = jnp.float32


def _ln(xb, g, b):
    m = jnp.mean(xb, axis=-1, keepdims=True)
    v = jnp.var(xb, axis=-1, keepdims=True)
    return (xb - m) * jax.lax.rsqrt(v + 1e-5) * g + b


def _k1_body(x_ref, g_ref, b_ref, w_ref, bc_ref, q_ref, k_ref, v_ref, gt_ref):
    xb = x_ref[:]
    ln = _ln(xb, g_ref[:], b_ref[:])
    out = jnp.dot(ln, w_ref[:], preferred_element_type=F32) + bc_ref[:]
    q_ref[:] = out[:, :D]
    k_ref[:] = out[:, D:D + HKV * HD]
    v_ref[:] = out[:, D + HKV * HD:D + 2 * HKV * HD]
    gt_ref[:] = jax.nn.sigmoid(out[:, D + 2 * HKV * HD:])


def _k1(x, ln1_g, ln1_b, Wcat, bcat):
    blk = 256
    return pl.pallas_call(
        _k1_body,
        grid=(S // blk,),
        compiler_params=pltpu.CompilerParams(dimension_semantics=("parallel",)),
        in_specs=[
            pl.BlockSpec((blk, D), lambda i: (i, 0)),
            pl.BlockSpec((1, D), lambda i: (0, 0)),
            pl.BlockSpec((1, D), lambda i: (0, 0)),
            pl.BlockSpec(Wcat.shape, lambda i: (0, 0)),
            pl.BlockSpec((1, Wcat.shape[1]), lambda i: (0, 0)),
        ],
        out_specs=[
            pl.BlockSpec((blk, D), lambda i: (i, 0)),
            pl.BlockSpec((blk, HKV * HD), lambda i: (i, 0)),
            pl.BlockSpec((blk, HKV * HD), lambda i: (i, 0)),
            pl.BlockSpec((blk, 128), lambda i: (i, 0)),
        ],
        out_shape=[
            jax.ShapeDtypeStruct((S, D), F32),
            jax.ShapeDtypeStruct((S, HKV * HD), F32),
            jax.ShapeDtypeStruct((S, HKV * HD), F32),
            jax.ShapeDtypeStruct((S, 128), F32),
        ],
    )(x, ln1_g, ln1_b, Wcat, bcat)


QC3 = 512


def _kb_body(q_ref, kf_ref, vf_ref, wk_ref, bk_ref, wv_ref, bv_ref,
             out_ref, idx_ref, ck_s, cv_s, impq_s):
    i = pl.program_id(1)
    nsteps = pl.num_programs(1)

    @pl.when(i == 0)
    def _():
        kr = kf_ref[0]
        vr = vf_ref[0]
        zero = jnp.zeros((1, HD), F32)

        def proj(r, w_ref, b_ref):
            t0 = jnp.dot(r, w_ref[:STRIDE * HD], preferred_element_type=F32)
            t1 = jnp.dot(r, w_ref[STRIDE * HD:], preferred_element_type=F32)
            return t0 + jnp.concatenate([t1[1:], zero], axis=0) + b_ref[:]

        ck_s[:] = proj(kr, wk_ref, bk_ref)
        cv_s[:] = proj(vr, wv_ref, bv_ref)

    ckm = ck_s[:]
    cvm = cv_s[:]
    qpos = i * QC3 + jax.lax.broadcasted_iota(jnp.int32, (QC3, 1), 0)
    nidx = jax.lax.broadcasted_iota(jnp.int32, (1, NCMP_PAD), 1)
    mask = qpos >= nidx * STRIDE + (L - 1)
    pad = nidx < NCMP

    cps = jnp.zeros((QC3, NCMP_PAD), F32)
    for hp in range(HPG):
        qh = q_ref[:, hp * HD:(hp + 1) * HD]
        s = jax.lax.dot_general(qh, ckm, (((1,), (1,)), ((), ())),
                                preferred_element_type=F32) * SCALE
        s = jnp.where(mask, s, -1e9)
        s = jnp.where(pad, s, -1e30)
        m = jnp.max(s, axis=-1, keepdims=True)
        p = jnp.exp(s - m)
        cp = p / jnp.sum(p, axis=-1, keepdims=True)
        out_ref[:, hp * HD:(hp + 1) * HD] = jnp.dot(
            cp, cvm, preferred_element_type=F32)
        cps = cps + cp

    rr = jax.lax.broadcasted_iota(jnp.int32, (NCMP_PAD, NBLK), 0)
    cc = jax.lax.broadcasted_iota(jnp.int32, (NCMP_PAD, NBLK), 1)
    P = jnp.where((rr // 2 == cc) & (rr < NCMP), 1.0, 0.0).astype(F32)
    folded = jnp.dot(cps, P, preferred_element_type=F32)
    nq = QC3 // L
    impq_s[pl.ds(i * nq, nq), :] = jnp.sum(
        folded.reshape(nq, L, NBLK), axis=1)

    @pl.when(i == nsteps - 1)
    def _():
        vals = impq_s[:]
        qb = jax.lax.broadcasted_iota(jnp.int32, (NBLK, NBLK), 0)
        mb = jax.lax.broadcasted_iota(jnp.int32, (NBLK, NBLK), 1)
        bonus = jnp.where((mb == qb) | (mb == 0), 1e6, 0.0).astype(F32)
        vals = jnp.where(qb >= mb, vals + bonus, -1e9)
        tcol = jax.lax.broadcasted_iota(jnp.int32, (NBLK, TOPN), 1)
        out = jnp.zeros((NBLK, TOPN), jnp.int32)
        for t in range(TOPN):
            m = jnp.argmax(vals, axis=1).astype(jnp.int32)
            out = jnp.where(tcol == t, m[:, None], out)
            vals = jnp.where(mb == m[:, None], -3e9, vals)
        idx_ref[0] = out


def _kb(q, kflat, vflat, Wck, bck, Wcv, bcv):
    return pl.pallas_call(
        _kb_body,
        grid=(HKV, S // QC3),
        compiler_params=pltpu.CompilerParams(
            dimension_semantics=("arbitrary", "arbitrary")),
        in_specs=[
            pl.BlockSpec((QC3, HPG * HD), lambda g, i: (i, g)),
            pl.BlockSpec((1, S // STRIDE, STRIDE * HD), lambda g, i: (g, 0, 0)),
            pl.BlockSpec((1, S // STRIDE, STRIDE * HD), lambda g, i: (g, 0, 0)),
            pl.BlockSpec(Wck.shape, lambda g, i: (0, 0)),
            pl.BlockSpec((1, HD), lambda g, i: (0, 0)),
            pl.BlockSpec(Wcv.shape, lambda g, i: (0, 0)),
            pl.BlockSpec((1, HD), lambda g, i: (0, 0)),
        ],
        out_specs=[
            pl.BlockSpec((QC3, HPG * HD), lambda g, i: (i, g)),
            pl.BlockSpec((1, NBLK, TOPN), lambda g, i: (g, 0, 0)),
        ],
        out_shape=[
            jax.ShapeDtypeStruct((S, D), F32),
            jax.ShapeDtypeStruct((HKV, NBLK, TOPN), jnp.int32),
        ],
        scratch_shapes=[
            pltpu.VMEM((NCMP_PAD, HD), F32),
            pltpu.VMEM((NCMP_PAD, HD), F32),
            pltpu.VMEM((NBLK, NBLK), F32),
        ],
    )(q, kflat, vflat, Wck, bck, Wcv, bcv)


QC = 256
QB5 = QC // L
WK = WIN + QC


def _kc_body(idx_ref, q_ref, kv_ref, sel_ref, win_ref, kv_scr):
    g = pl.program_id(0)
    i = pl.program_id(1)

    rows = jax.lax.broadcasted_iota(jnp.int32, (HPG * L, 1), 0)
    jcol = jax.lax.broadcasted_iota(jnp.int32, (1, TOPN * L), 1)
    jmod = jcol % L

    for qq in range(QB5):
        qb = i * QB5 + qq
        base = g * NBLK * TOPN + qb * TOPN
        qpos = qb * L + rows % L

        colpos = jmod
        for t in range(TOPN):
            it = idx_ref[base + t]
            kv_scr[qq * TOPN * L + t * L:qq * TOPN * L + (t + 1) * L, :] = (
                kv_ref[0, pl.ds(it * L, L), :])
            colpos = colpos + jnp.where(jcol // L == t, it * L, 0)
        mask = colpos <= qpos

        ks = kv_scr[qq * TOPN * L:(qq + 1) * TOPN * L, :HD]
        vs = kv_scr[qq * TOPN * L:(qq + 1) * TOPN * L, HD:]
        q4 = jnp.concatenate(
            [q_ref[qq * L:(qq + 1) * L, hp * HD:(hp + 1) * HD]
             for hp in range(HPG)], axis=0)
        s = jax.lax.dot_general(q4, ks, (((1,), (1,)), ((), ())),
                                preferred_element_type=F32) * SCALE
        s = jnp.where(mask, s, -1e9)
        m = jnp.max(s, axis=-1, keepdims=True)
        p = jnp.exp(s - m)
        r = 1.0 / jnp.sum(p, axis=-1, keepdims=True)
        o = jnp.dot(p, vs, preferred_element_type=F32) * r
        for hp in range(HPG):
            sel_ref[qq * L:(qq + 1) * L, hp * HD:(hp + 1) * HD] = (
                o[hp * L:(hp + 1) * L, :])

    s0 = jnp.maximum(i - WIN // QC, 0) * QC
    kvw = kv_ref[0, pl.ds(s0, WK), :]
    kw = kvw[:, :HD]
    vw = kvw[:, HD:]
    qpos = i * QC + jax.lax.broadcasted_iota(jnp.int32, (QC, 1), 0)
    kpos = s0 + jax.lax.broadcasted_iota(jnp.int32, (1, WK), 1)
    wmask = (qpos >= kpos) & (qpos - kpos < WIN)
    for hp in range(HPG):
        qh = q_ref[:, hp * HD:(hp + 1) * HD]
        s = jax.lax.dot_general(qh, kw, (((1,), (1,)), ((), ())),
                                preferred_element_type=F32) * SCALE
        s = jnp.where(wmask, s, -1e9)
        m = jnp.max(s, axis=-1, keepdims=True)
        p = jnp.exp(s - m)
        r = 1.0 / jnp.sum(p, axis=-1, keepdims=True)
        win_ref[:, hp * HD:(hp + 1) * HD] = jnp.dot(
            p, vw, preferred_element_type=F32) * r


def _kc(top_idx_flat, q, kvh):
    grid_spec = pltpu.PrefetchScalarGridSpec(
        num_scalar_prefetch=1,
        grid=(HKV, S // QC),
        in_specs=[
            pl.BlockSpec((QC, HPG * HD), lambda g, i, *_: (i, g)),
            pl.BlockSpec((1, S, 2 * HD), lambda g, i, *_: (g, 0, 0)),
        ],
        out_specs=[
            pl.BlockSpec((QC, HPG * HD), lambda g, i, *_: (i, g)),
            pl.BlockSpec((QC, HPG * HD), lambda g, i, *_: (i, g)),
        ],
        scratch_shapes=[
            pltpu.VMEM((QB5 * TOPN * L, 2 * HD), F32),
        ],
    )
    return pl.pallas_call(
        _kc_body,
        grid_spec=grid_spec,
        compiler_params=pltpu.CompilerParams(
            dimension_semantics=("parallel", "parallel")),
        out_shape=[
            jax.ShapeDtypeStruct((S, D), F32),
            jax.ShapeDtypeStruct((S, D), F32),
        ],
    )(top_idx_flat, q, kvh)


def _k7_body(x_ref, cmp_ref, sel_ref, win_ref, g_ref, wo_ref,
             ln2g_ref, ln2b_ref, w1_ref, b1_ref, w2_ref, b2_ref, out_ref):
    gts = g_ref[:]
    rr = jax.lax.broadcasted_iota(jnp.int32, (128, D), 0)
    cc = jax.lax.broadcasted_iota(jnp.int32, (128, D), 1)
    head3 = 3 * (cc // HD)
    e0 = jnp.where(rr == head3, 1.0, 0.0).astype(F32)
    e1 = jnp.where(rr == head3 + 1, 1.0, 0.0).astype(F32)
    e2 = jnp.where(rr == head3 + 2, 1.0, 0.0).astype(F32)
    comb = (cmp_ref[:] * jnp.dot(gts, e0, preferred_element_type=F32)
            + sel_ref[:] * jnp.dot(gts, e1, preferred_element_type=F32)
            + win_ref[:] * jnp.dot(gts, e2, preferred_element_type=F32))
    x1 = x_ref[:] + jnp.dot(comb, wo_ref[:], preferred_element_type=F32)
    ln = _ln(x1, ln2g_ref[:], ln2b_ref[:])
    h = jax.nn.gelu(jnp.dot(ln, w1_ref[:], preferred_element_type=F32) + b1_ref[:])
    out_ref[:] = x1 + jnp.dot(h, w2_ref[:], preferred_element_type=F32) + b2_ref[:]


def _k7(x, out_cmp, out_sel, out_win, gates, Wo, ln2_g, ln2_b, W1, b1, W2, b2):
    blk = 256
    return pl.pallas_call(
        _k7_body,
        grid=(S // blk,),
        compiler_params=pltpu.CompilerParams(dimension_semantics=("parallel",)),
        in_specs=[
            pl.BlockSpec((blk, D), lambda i: (i, 0)),
            pl.BlockSpec((blk, D), lambda i: (i, 0)),
            pl.BlockSpec((blk, D), lambda i: (i, 0)),
            pl.BlockSpec((blk, D), lambda i: (i, 0)),
            pl.BlockSpec((blk, 128), lambda i: (i, 0)),
            pl.BlockSpec((D, D), lambda i: (0, 0)),
            pl.BlockSpec((1, D), lambda i: (0, 0)),
            pl.BlockSpec((1, D), lambda i: (0, 0)),
            pl.BlockSpec((D, 4 * D), lambda i: (0, 0)),
            pl.BlockSpec((1, 4 * D), lambda i: (0, 0)),
            pl.BlockSpec((4 * D, D), lambda i: (0, 0)),
            pl.BlockSpec((1, D), lambda i: (0, 0)),
        ],
        out_specs=pl.BlockSpec((blk, D), lambda i: (i, 0)),
        out_shape=jax.ShapeDtypeStruct((S, D), F32),
    )(x, out_cmp, out_sel, out_win, gates, Wo, ln2_g, ln2_b, W1, b1, W2, b2)


@jax.jit
def _run(x, ln1_g, ln1_b, Wq, Wk, Wv, Wck, bck, Wcv, bcv, Wg, bg, Wo,
         ln2_g, ln2_b, W1, b1, W2, b2):
    x2d = x[0]
    Wg_pad = jnp.pad(Wg, ((0, 0), (0, 128 - 3 * H)))
    bcat = jnp.concatenate(
        [jnp.zeros((D + 2 * HKV * HD,), F32), bg,
         jnp.zeros((128 - 3 * H,), F32)])[None]
    Wcat = jnp.concatenate([Wq, Wk, Wv, Wg_pad], axis=1)

    q, k, v, gates = _k1(x2d, ln1_g[None], ln1_b[None], Wcat, bcat)

    kh = k.reshape(S, HKV, HD).transpose(1, 0, 2)
    vh = v.reshape(S, HKV, HD).transpose(1, 0, 2)
    kvh = jnp.concatenate([kh, vh], axis=-1)
    kf = kh.reshape(HKV, S // STRIDE, STRIDE * HD)
    vf = vh.reshape(HKV, S // STRIDE, STRIDE * HD)

    out_cmp, top_idx = _kb(q, kf, vf, Wck, bck[None], Wcv, bcv[None])
    out_sel, out_win = _kc(top_idx.reshape(-1), q, kvh)
    out = _k7(x2d, out_cmp, out_sel, out_win, gates, Wo,
              ln2_g[None], ln2_b[None], W1, b1[None], W2, b2[None])
    return out[None]


def kernel(x, ln1_g, ln1_b, Wq, Wk, Wv, Wck, bck, Wcv, bcv, Wg, bg, Wo,
           ln2_g, ln2_b, W1, b1, W2, b2):
    return _run(x, ln1_g, ln1_b, Wq, Wk, Wv, Wck, bck, Wcv, bcv, Wg, bg, Wo,
                ln2_g, ln2_b, W1, b1, W2, b2)
